# baseline (speedup 1.0000x reference)
import jax
import jax.numpy as jnp
from jax.experimental import pallas as pl
from jax.experimental.pallas import tpu as pltpu

_N = 100
_IN = 128
_OUT = 128
_E = 16
_N_BLK = 2


def _combine_kernel(gw_ref, experts_ref, biases_ref, cw_ref, cb_ref):
    g = jax.nn.softmax(gw_ref[...], axis=-1)
    cw = jnp.dot(g, experts_ref[...], preferred_element_type=jnp.float32)
    cw_ref[...] = cw.astype(jnp.bfloat16)
    cb_ref[...] = jnp.dot(g, biases_ref[...], preferred_element_type=jnp.float32)


def _apply_kernel(x_ref, w_ref, b_ref, out_ref):
    for k in range(_N_BLK):
        xb = x_ref[k].astype(jnp.bfloat16)
        y = jnp.dot(xb, w_ref[k], preferred_element_type=jnp.float32)
        out_ref[k] = y + b_ref[k]


def kernel(x, gate_weights, experts, expert_biases):
    batch = x.shape[0]
    experts2 = experts.reshape(_E, _IN * _OUT)

    cw2, cb = pl.pallas_call(
        _combine_kernel,
        out_shape=(
            jax.ShapeDtypeStruct((_N, _IN * _OUT), jnp.bfloat16),
            jax.ShapeDtypeStruct((_N, _OUT), jnp.float32),
        ),
    )(gate_weights, experts2, expert_biases)
    cw = cw2.reshape(_N, _IN, _OUT)
    cb3 = cb.reshape(_N, 1, _OUT)

    xt = jnp.swapaxes(x, 0, 1)
    out_t = pl.pallas_call(
        _apply_kernel,
        grid=(_N // _N_BLK,),
        in_specs=[
            pl.BlockSpec((_N_BLK, batch, _IN), lambda n: (n, 0, 0)),
            pl.BlockSpec((_N_BLK, _IN, _OUT), lambda n: (n, 0, 0)),
            pl.BlockSpec((_N_BLK, 1, _OUT), lambda n: (n, 0, 0)),
        ],
        out_specs=pl.BlockSpec((_N_BLK, batch, _OUT), lambda n: (n, 0, 0)),
        out_shape=jax.ShapeDtypeStruct((_N, batch, _OUT), jnp.float32),
        compiler_params=pltpu.CompilerParams(
            dimension_semantics=("parallel",),
        ),
    )(xt, cw, cb3)
    return jnp.swapaxes(out_t, 0, 1)


# speedup vs baseline: 1.0205x; 1.0205x over previous
import jax
import jax.numpy as jnp
from jax.experimental import pallas as pl
from jax.experimental.pallas import tpu as pltpu

_N = 100
_IN = 128
_OUT = 128
_E = 16
_N_BLK = 5


def _combine_kernel(gw_ref, experts_ref, biases_ref, cw_ref, cb_ref):
    g = jax.nn.softmax(gw_ref[...], axis=-1)
    cw = jnp.dot(g, experts_ref[...], preferred_element_type=jnp.float32)
    cw_ref[...] = cw.astype(jnp.bfloat16)
    cb_ref[...] = jnp.dot(g, biases_ref[...], preferred_element_type=jnp.float32)


def _apply_kernel(x_ref, w_ref, b_ref, out_ref):
    for k in range(_N_BLK):
        xb = x_ref[k].astype(jnp.bfloat16)
        y = jnp.dot(xb, w_ref[k], preferred_element_type=jnp.float32)
        out_ref[k] = y + b_ref[k]


def kernel(x, gate_weights, experts, expert_biases):
    batch = x.shape[0]
    experts2 = experts.reshape(_E, _IN * _OUT)

    cw2, cb = pl.pallas_call(
        _combine_kernel,
        out_shape=(
            jax.ShapeDtypeStruct((_N, _IN * _OUT), jnp.bfloat16),
            jax.ShapeDtypeStruct((_N, _OUT), jnp.float32),
        ),
    )(gate_weights, experts2, expert_biases)
    cw = cw2.reshape(_N, _IN, _OUT)
    cb3 = cb.reshape(_N, 1, _OUT)

    xt = jnp.swapaxes(x, 0, 1)
    out_t = pl.pallas_call(
        _apply_kernel,
        grid=(_N // _N_BLK,),
        in_specs=[
            pl.BlockSpec((_N_BLK, batch, _IN), lambda n: (n, 0, 0)),
            pl.BlockSpec((_N_BLK, _IN, _OUT), lambda n: (n, 0, 0)),
            pl.BlockSpec((_N_BLK, 1, _OUT), lambda n: (n, 0, 0)),
        ],
        out_specs=pl.BlockSpec((_N_BLK, batch, _OUT), lambda n: (n, 0, 0)),
        out_shape=jax.ShapeDtypeStruct((_N, batch, _OUT), jnp.float32),
        compiler_params=pltpu.CompilerParams(
            dimension_semantics=("parallel",),
        ),
    )(xt, cw, cb3)
    return jnp.swapaxes(out_t, 0, 1)


# speedup vs baseline: 1.0910x; 1.0690x over previous
import jax
import jax.numpy as jnp
from jax.experimental import pallas as pl
from jax.experimental.pallas import tpu as pltpu

_N = 100
_IN = 128
_OUT = 128
_E = 16
_N_BLK = 5


def _fused_kernel(x_ref, gw_ref, experts_ref, biases_ref, out_ref):
    g = jax.nn.softmax(gw_ref[0], axis=-1)
    gb = g.astype(jnp.bfloat16)
    eb = experts_ref[...].astype(jnp.bfloat16)
    cw = jnp.dot(gb, eb, preferred_element_type=jnp.float32)
    cwb = cw.astype(jnp.bfloat16).reshape(_N_BLK, _IN, _OUT)
    cb = jnp.dot(g, biases_ref[...], preferred_element_type=jnp.float32)
    for k in range(_N_BLK):
        xb = x_ref[k].astype(jnp.bfloat16)
        y = jnp.dot(xb, cwb[k], preferred_element_type=jnp.float32)
        out_ref[k] = y + cb[k : k + 1, :]


def kernel(x, gate_weights, experts, expert_biases):
    batch = x.shape[0]
    experts2 = experts.reshape(_E, _IN * _OUT)
    gw3 = gate_weights.reshape(_N // _N_BLK, _N_BLK, _E)

    xt = jnp.swapaxes(x, 0, 1)
    out_t = pl.pallas_call(
        _fused_kernel,
        grid=(_N // _N_BLK,),
        in_specs=[
            pl.BlockSpec((_N_BLK, batch, _IN), lambda n: (n, 0, 0)),
            pl.BlockSpec((1, _N_BLK, _E), lambda n: (n, 0, 0)),
            pl.BlockSpec((_E, _IN * _OUT), lambda n: (0, 0)),
            pl.BlockSpec((_E, _OUT), lambda n: (0, 0)),
        ],
        out_specs=pl.BlockSpec((_N_BLK, batch, _OUT), lambda n: (n, 0, 0)),
        out_shape=jax.ShapeDtypeStruct((_N, batch, _OUT), jnp.float32),
        compiler_params=pltpu.CompilerParams(
            dimension_semantics=("parallel",),
        ),
    )(xt, gw3, experts2, expert_biases)
    return jnp.swapaxes(out_t, 0, 1)


# speedup vs baseline: 1.1152x; 1.0222x over previous
import jax
import jax.numpy as jnp
from jax.experimental import pallas as pl
from jax.experimental.pallas import tpu as pltpu

_N = 100
_IN = 128
_OUT = 128
_E = 16
_N_BLK = 5


def _fused_kernel(x_ref, gw_ref, experts_ref, biases_ref, out_ref):
    g = jax.nn.softmax(gw_ref[0], axis=-1)
    gb = g.astype(jnp.bfloat16)
    eb = experts_ref[...].astype(jnp.bfloat16).reshape(_E, _IN * _OUT)
    cw = jnp.dot(gb, eb, preferred_element_type=jnp.float32)
    cwb = cw.astype(jnp.bfloat16).reshape(_N_BLK, _IN, _OUT)
    cb = jnp.dot(g, biases_ref[...], preferred_element_type=jnp.float32)
    for k in range(_N_BLK):
        xb = x_ref[k].astype(jnp.bfloat16)
        y = jnp.dot(xb, cwb[k], preferred_element_type=jnp.float32)
        out_ref[k] = y + cb[k : k + 1, :]


def kernel(x, gate_weights, experts, expert_biases):
    batch = x.shape[0]
    gw3 = gate_weights.reshape(_N // _N_BLK, _N_BLK, _E)

    xt = jnp.swapaxes(x, 0, 1)
    out_t = pl.pallas_call(
        _fused_kernel,
        grid=(_N // _N_BLK,),
        in_specs=[
            pl.BlockSpec((_N_BLK, batch, _IN), lambda n: (n, 0, 0)),
            pl.BlockSpec((1, _N_BLK, _E), lambda n: (n, 0, 0)),
            pl.BlockSpec((_E, _IN, _OUT), lambda n: (0, 0, 0)),
            pl.BlockSpec((_E, _OUT), lambda n: (0, 0)),
        ],
        out_specs=pl.BlockSpec((_N_BLK, batch, _OUT), lambda n: (n, 0, 0)),
        out_shape=jax.ShapeDtypeStruct((_N, batch, _OUT), jnp.float32),
        compiler_params=pltpu.CompilerParams(
            dimension_semantics=("parallel",),
        ),
    )(xt, gw3, experts, expert_biases)
    return jnp.swapaxes(out_t, 0, 1)
